# baseline (speedup 1.0000x reference)
import functools

import jax
import jax.numpy as jnp
from jax import lax
from jax.experimental import pallas as pl
from jax.experimental.pallas import tpu as pltpu
from jax.experimental.pallas import tpu_sc as plsc

VOCAB = 1000000
EMB = 64
MAXPOS = 512
B = 1024
L = 512

N = B * L
NC = 2
NS = 16
NW = NC * NS
B_PER_W = B // NW
CHUNK = 256
LANES = 16
ET = EMB // 8
NLT = CHUNK // 128
TBW = NLT * 8 * 128

_mesh = plsc.VectorSubcoreMesh(core_axis_name="c", subcore_axis_name="s")


TOT = B_PER_W * 2


def _chunk_start(c, base_b):
    bb = base_b + c // 2
    half = c % 2
    return bb, half, bb * L + half * CHUNK


@functools.partial(
    pl.kernel,
    mesh=_mesh,
    out_type=jax.ShapeDtypeStruct((B, ET, L // 128, 8, 128), jnp.float32),
    scratch_types=[
        pltpu.VMEM((MAXPOS, EMB), jnp.float32),
        pltpu.VMEM((CHUNK,), jnp.int32),
        pltpu.VMEM((CHUNK,), jnp.int32),
        pltpu.VMEM((2 * CHUNK,), jnp.int32),
        pltpu.VMEM((2 * CHUNK,), jnp.int32),
        pltpu.VMEM((2 * CHUNK, 32), jnp.float32),
        pltpu.VMEM((2 * CHUNK, 32), jnp.float32),
        pltpu.VMEM((ET, NLT, 8, 128), jnp.float32),
        pltpu.SemaphoreType.DMA,
        pltpu.SemaphoreType.DMA,
        pltpu.SemaphoreType.DMA,
        pltpu.SemaphoreType.DMA,
    ],
    compiler_params=pltpu.CompilerParams(
        use_tc_tiling_on_sc=False, needs_layout_passes=False,
        disable_bounds_checks=True,
    ),
)
def _embed(ids_hbm, word_hbm, pos_hbm, out_hbm, pos_v, idx0, idx1, pidx0,
           pidx1, rows0, rows1, tb_v, gsem0, gsem1, isem, osem):
    wid = lax.axis_index("s") * NC + lax.axis_index("c")
    base_b = wid * B_PER_W
    idx_bufs = (idx0, idx1)
    pidx_bufs = (pidx0, pidx1)
    rows_bufs = (rows0, rows1)
    gsems = (gsem0, gsem1)

    pltpu.sync_copy(pos_hbm, pos_v)

    viota16 = lax.iota(jnp.int32, LANES)
    e_rows = tuple((jnp.full((LANES,), c * LANES, jnp.int32) + viota16)
                   // 8 for c in range(EMB // LANES))
    e_subs = tuple((jnp.full((LANES,), c * LANES, jnp.int32) + viota16) % 8
                   for c in range(EMB // LANES))


    viota16 = lax.iota(jnp.int32, LANES)
    _srcv = viota16 // 2
    _lowb = viota16 % 2

    def expand_idx(idx_ref, pidx_ref):
        for v in range(2 * CHUNK // LANES):
            vids = plsc.load_gather(idx_ref, [_srcv + v * (LANES // 2)])
            pidx_ref[pl.ds(v * LANES, LANES)] = vids * 4 + _lowb

    _, _, s0 = _chunk_start(0, base_b)
    pltpu.sync_copy(ids_hbm.at[pl.ds(s0, CHUNK)], idx0)
    expand_idx(idx0, pidx0)
    pltpu.async_copy(word_hbm.at[pidx0], rows0, gsem0)
    _, _, s1 = _chunk_start(1, base_b)
    pltpu.async_copy(ids_hbm.at[pl.ds(s1, CHUNK)], idx1, isem)

    def one_chunk(c, p):
        bb, _, _ = _chunk_start(c, base_b)
        half = p
        pltpu.make_async_copy(ids_hbm.at[pl.ds(0, CHUNK)], idx_bufs[1 - p],
                              isem).wait()
        expand_idx(idx_bufs[1 - p], pidx_bufs[1 - p])
        pltpu.async_copy(word_hbm.at[pidx_bufs[1 - p]], rows_bufs[1 - p],
                         gsems[1 - p])
        pltpu.make_async_copy(word_hbm.at[pl.ds(0, 2 * CHUNK)], rows_bufs[p],
                              gsems[p]).wait()
        rows = rows_bufs[p]
        cn2 = jnp.minimum(c + 2, TOT - 1)
        _, _, sn2 = _chunk_start(cn2, base_b)
        pltpu.async_copy(ids_hbm.at[pl.ds(sn2, CHUNK)], idx_bufs[p], isem)

        @pl.when(c > 0)
        def _():
            pltpu.make_async_copy(tb_v, out_hbm.at[0, :, pl.ds(0, NLT)],
                                  osem).wait()

        @plsc.parallel_loop(0, CHUNK, 1, unroll=4)
        def asm_body(l):
            ltv = jnp.full((LANES,), 0, jnp.int32) + l // 128
            lmv = jnp.full((LANES,), 0, jnp.int32) + l % 128
            pl_row = half * CHUNK + l
            for cg in range(EMB // LANES):
                vals = rows[2 * l + cg // 2, pl.ds((cg % 2) * LANES, LANES)]
                vals = vals + pos_v[pl_row, pl.ds(cg * LANES, LANES)]
                plsc.store_scatter(tb_v, [e_rows[cg], ltv, e_subs[cg], lmv],
                                   vals)

        pltpu.async_copy(tb_v, out_hbm.at[bb, :, pl.ds(half * NLT, NLT)],
                         osem)

    def pair_body(g2, carry):
        one_chunk(2 * g2, 0)
        one_chunk(2 * g2 + 1, 1)
        return carry

    lax.fori_loop(0, B_PER_W, pair_body, 0)
    pltpu.make_async_copy(tb_v, out_hbm.at[0, :, pl.ds(0, NLT)], osem).wait()
    pltpu.make_async_copy(word_hbm.at[pl.ds(0, 2 * CHUNK)], rows_bufs[0],
                          gsems[0]).wait()
    pltpu.make_async_copy(ids_hbm.at[pl.ds(0, CHUNK)], idx_bufs[1],
                          isem).wait()


TBLOCK = 8192


def _tr_body(in_ref, out_ref):
    out_ref[:, :EMB] = in_ref[...].T


_transpose = pl.pallas_call(
    _tr_body,
    grid=(pl.cdiv(VOCAB, TBLOCK),),
    in_specs=[pl.BlockSpec((EMB, TBLOCK), lambda j: (0, j))],
    out_specs=pl.BlockSpec((TBLOCK, 128), lambda j: (j, 0)),
    out_shape=jax.ShapeDtypeStruct((VOCAB, 128), jnp.float32),
)


def kernel(input_ids, word_table, pos_table):
    wt4 = _transpose(word_table.T).reshape(4 * VOCAB, 32)
    ids_flat = input_ids.reshape(N).astype(jnp.int32)
    out5 = _embed(ids_flat, wt4, pos_table)
    return out5.transpose(0, 2, 4, 1, 3).reshape(B, L, EMB)


# speedup vs baseline: 1.9918x; 1.9918x over previous
import functools

import jax
import jax.numpy as jnp
from jax import lax
from jax.experimental import pallas as pl
from jax.experimental.pallas import tpu as pltpu
from jax.experimental.pallas import tpu_sc as plsc

VOCAB = 1000000
EMB = 64
MAXPOS = 512
B = 1024
L = 512

N = B * L
NC = 2
NS = 16
NW = NC * NS
B_PER_W = B // NW
CHUNK = 256
LANES = 16
ET = EMB // 8
NLT = CHUNK // 128
TBW = NLT * 8 * 128

_mesh = plsc.VectorSubcoreMesh(core_axis_name="c", subcore_axis_name="s")


TOT = B_PER_W * 2


def _chunk_start(c, base_b):
    bb = base_b + c // 2
    half = c % 2
    return bb, half, bb * L + half * CHUNK


@functools.partial(
    pl.kernel,
    mesh=_mesh,
    out_type=jax.ShapeDtypeStruct((B, ET, L // 128, 8, 128), jnp.float32),
    scratch_types=[
        pltpu.VMEM((MAXPOS, EMB), jnp.float32),
        pltpu.VMEM((CHUNK,), jnp.int32),
        pltpu.VMEM((CHUNK,), jnp.int32),
        pltpu.VMEM((2 * CHUNK,), jnp.int32),
        pltpu.VMEM((2 * CHUNK,), jnp.int32),
        pltpu.VMEM((2 * CHUNK, 32), jnp.float32),
        pltpu.VMEM((2 * CHUNK, 32), jnp.float32),
        pltpu.VMEM((ET, NLT, 8, 129), jnp.float32),
        pltpu.SemaphoreType.DMA,
        pltpu.SemaphoreType.DMA,
        pltpu.SemaphoreType.DMA,
        pltpu.SemaphoreType.DMA,
    ],
    compiler_params=pltpu.CompilerParams(
        use_tc_tiling_on_sc=False, needs_layout_passes=False,
        disable_bounds_checks=True,
    ),
)
def _embed(ids_hbm, word_hbm, pos_hbm, out_hbm, pos_v, idx0, idx1, pidx0,
           pidx1, rows0, rows1, tb_v, gsem0, gsem1, isem, osem):
    wid = lax.axis_index("s") * NC + lax.axis_index("c")
    base_b = wid * B_PER_W
    idx_bufs = (idx0, idx1)
    pidx_bufs = (pidx0, pidx1)
    rows_bufs = (rows0, rows1)
    gsems = (gsem0, gsem1)

    pltpu.sync_copy(pos_hbm, pos_v)

    viota16 = lax.iota(jnp.int32, LANES)
    e_rows = tuple((jnp.full((LANES,), c * LANES, jnp.int32) + viota16)
                   // 8 for c in range(EMB // LANES))
    e_subs = tuple((jnp.full((LANES,), c * LANES, jnp.int32) + viota16) % 8
                   for c in range(EMB // LANES))


    viota16 = lax.iota(jnp.int32, LANES)
    _srcv = viota16 // 2
    _lowb = viota16 % 2

    def expand_idx(idx_ref, pidx_ref):
        for v in range(2 * CHUNK // LANES):
            vids = plsc.load_gather(idx_ref, [_srcv + v * (LANES // 2)])
            pidx_ref[pl.ds(v * LANES, LANES)] = vids * 4 + _lowb

    _, _, s0 = _chunk_start(0, base_b)
    pltpu.sync_copy(ids_hbm.at[pl.ds(s0, CHUNK)], idx0)
    expand_idx(idx0, pidx0)
    pltpu.async_copy(word_hbm.at[pidx0], rows0, gsem0)
    _, _, s1 = _chunk_start(1, base_b)
    pltpu.async_copy(ids_hbm.at[pl.ds(s1, CHUNK)], idx1, isem)

    def one_chunk(c, p):
        bb, _, _ = _chunk_start(c, base_b)
        half = p
        pltpu.make_async_copy(ids_hbm.at[pl.ds(0, CHUNK)], idx_bufs[1 - p],
                              isem).wait()
        expand_idx(idx_bufs[1 - p], pidx_bufs[1 - p])
        pltpu.async_copy(word_hbm.at[pidx_bufs[1 - p]], rows_bufs[1 - p],
                         gsems[1 - p])
        pltpu.make_async_copy(word_hbm.at[pl.ds(0, 2 * CHUNK)], rows_bufs[p],
                              gsems[p]).wait()
        rows = rows_bufs[p]
        cn2 = jnp.minimum(c + 2, TOT - 1)
        _, _, sn2 = _chunk_start(cn2, base_b)
        pltpu.async_copy(ids_hbm.at[pl.ds(sn2, CHUNK)], idx_bufs[p], isem)

        @pl.when(c > 0)
        def _():
            pltpu.make_async_copy(tb_v.at[:, :, :, pl.ds(0, 128)],
                                  out_hbm.at[0, :, pl.ds(0, NLT)],
                                  osem).wait()

        @plsc.parallel_loop(0, CHUNK, 1, unroll=4)
        def asm_body(l):
            ltv = jnp.full((LANES,), 0, jnp.int32) + l // 128
            lmv = jnp.full((LANES,), 0, jnp.int32) + l % 128
            pl_row = half * CHUNK + l
            for cg in range(EMB // LANES):
                vals = rows[2 * l + cg // 2, pl.ds((cg % 2) * LANES, LANES)]
                vals = vals + pos_v[pl_row, pl.ds(cg * LANES, LANES)]
                plsc.store_scatter(tb_v, [e_rows[cg], ltv, e_subs[cg], lmv],
                                   vals)

        pltpu.async_copy(tb_v.at[:, :, :, pl.ds(0, 128)],
                         out_hbm.at[bb, :, pl.ds(half * NLT, NLT)], osem)

    def pair_body(g2, carry):
        one_chunk(2 * g2, 0)
        one_chunk(2 * g2 + 1, 1)
        return carry

    lax.fori_loop(0, B_PER_W, pair_body, 0)
    pltpu.make_async_copy(tb_v.at[:, :, :, pl.ds(0, 128)],
                          out_hbm.at[0, :, pl.ds(0, NLT)], osem).wait()
    pltpu.make_async_copy(word_hbm.at[pl.ds(0, 2 * CHUNK)], rows_bufs[0],
                          gsems[0]).wait()
    pltpu.make_async_copy(ids_hbm.at[pl.ds(0, CHUNK)], idx_bufs[1],
                          isem).wait()


TBLOCK = 8192


def _tr_body(in_ref, out_ref):
    out_ref[:, :EMB] = in_ref[...].T


_transpose = pl.pallas_call(
    _tr_body,
    grid=(pl.cdiv(VOCAB, TBLOCK),),
    in_specs=[pl.BlockSpec((EMB, TBLOCK), lambda j: (0, j))],
    out_specs=pl.BlockSpec((TBLOCK, 128), lambda j: (j, 0)),
    out_shape=jax.ShapeDtypeStruct((VOCAB, 128), jnp.float32),
)


def kernel(input_ids, word_table, pos_table):
    wt4 = _transpose(word_table.T).reshape(4 * VOCAB, 32)
    ids_flat = input_ids.reshape(N).astype(jnp.int32)
    out5 = _embed(ids_flat, wt4, pos_table)
    return out5.transpose(0, 2, 4, 1, 3).reshape(B, L, EMB)


# speedup vs baseline: 2.0876x; 1.0481x over previous
import functools

import jax
import jax.numpy as jnp
from jax import lax
from jax.experimental import pallas as pl
from jax.experimental.pallas import tpu as pltpu
from jax.experimental.pallas import tpu_sc as plsc

VOCAB = 1000000
EMB = 64
MAXPOS = 512
B = 1024
L = 512

N = B * L
NC = 2
NS = 16
NW = NC * NS
B_PER_W = B // NW
CHUNK = 256
LANES = 16
ET = EMB // 8
NLT = CHUNK // 128
TBW = NLT * 8 * 128

_mesh = plsc.VectorSubcoreMesh(core_axis_name="c", subcore_axis_name="s")


TOT = B_PER_W * 2


def _chunk_start(c, base_b):
    bb = base_b + c // 2
    half = c % 2
    return bb, half, bb * L + half * CHUNK


@functools.partial(
    pl.kernel,
    mesh=_mesh,
    out_type=jax.ShapeDtypeStruct((B, ET, L // 128, 8, 128), jnp.float32),
    scratch_types=[
        pltpu.VMEM((MAXPOS, EMB), jnp.float32),
        pltpu.VMEM((CHUNK,), jnp.int32),
        pltpu.VMEM((CHUNK,), jnp.int32),
        pltpu.VMEM((2 * CHUNK,), jnp.int32),
        pltpu.VMEM((2 * CHUNK,), jnp.int32),
        pltpu.VMEM((2 * CHUNK, 32), jnp.float32),
        pltpu.VMEM((2 * CHUNK, 32), jnp.float32),
        pltpu.VMEM((ET, NLT, 8, 129), jnp.float32),
        pltpu.SemaphoreType.DMA,
        pltpu.SemaphoreType.DMA,
        pltpu.SemaphoreType.DMA,
        pltpu.SemaphoreType.DMA,
    ],
    compiler_params=pltpu.CompilerParams(
        use_tc_tiling_on_sc=False, needs_layout_passes=False,
        disable_bounds_checks=True,
    ),
)
def _embed(ids_hbm, word_hbm, pos_hbm, out_hbm, pos_v, idx0, idx1, pidx0,
           pidx1, rows0, rows1, tb_v, gsem0, gsem1, isem, osem):
    wid = lax.axis_index("s") * NC + lax.axis_index("c")
    base_b = wid * B_PER_W
    idx_bufs = (idx0, idx1)
    pidx_bufs = (pidx0, pidx1)
    rows_bufs = (rows0, rows1)
    gsems = (gsem0, gsem1)

    pltpu.sync_copy(pos_hbm, pos_v)

    viota16 = lax.iota(jnp.int32, LANES)
    e_rows = tuple((jnp.full((LANES,), c * LANES, jnp.int32) + viota16)
                   // 8 for c in range(EMB // LANES))
    e_subs = tuple((jnp.full((LANES,), c * LANES, jnp.int32) + viota16) % 8
                   for c in range(EMB // LANES))


    viota16 = lax.iota(jnp.int32, LANES)
    _srcv = viota16 // 2
    _lowb = viota16 % 2

    def expand_idx(idx_ref, pidx_ref):
        for v in range(2 * CHUNK // LANES):
            vids = plsc.load_gather(idx_ref, [_srcv + v * (LANES // 2)])
            pidx_ref[pl.ds(v * LANES, LANES)] = vids * 4 + _lowb

    _, _, s0 = _chunk_start(0, base_b)
    pltpu.sync_copy(ids_hbm.at[pl.ds(s0, CHUNK)], idx0)
    expand_idx(idx0, pidx0)
    pltpu.async_copy(word_hbm.at[pidx0], rows0, gsem0)
    _, _, s1 = _chunk_start(1, base_b)
    pltpu.async_copy(ids_hbm.at[pl.ds(s1, CHUNK)], idx1, isem)

    def one_chunk(c, p):
        bb, _, _ = _chunk_start(c, base_b)
        half = p
        pltpu.make_async_copy(ids_hbm.at[pl.ds(0, CHUNK)], idx_bufs[1 - p],
                              isem).wait()
        expand_idx(idx_bufs[1 - p], pidx_bufs[1 - p])
        pltpu.async_copy(word_hbm.at[pidx_bufs[1 - p]], rows_bufs[1 - p],
                         gsems[1 - p])
        pltpu.make_async_copy(word_hbm.at[pl.ds(0, 2 * CHUNK)], rows_bufs[p],
                              gsems[p]).wait()
        rows = rows_bufs[p]
        cn2 = jnp.minimum(c + 2, TOT - 1)
        _, _, sn2 = _chunk_start(cn2, base_b)
        pltpu.async_copy(ids_hbm.at[pl.ds(sn2, CHUNK)], idx_bufs[p], isem)

        @pl.when(c > 0)
        def _():
            pltpu.make_async_copy(tb_v.at[:, :, :, pl.ds(0, 128)],
                                  out_hbm.at[0, :, pl.ds(0, NLT)],
                                  osem).wait()

        @plsc.parallel_loop(0, CHUNK, 1, unroll=4)
        def asm_body(l):
            ltv = jnp.full((LANES,), 0, jnp.int32) + l // 128
            lmv = jnp.full((LANES,), 0, jnp.int32) + l % 128
            pl_row = half * CHUNK + l
            for cg in range(EMB // LANES):
                vals = rows[2 * l + cg // 2, pl.ds((cg % 2) * LANES, LANES)]
                vals = vals + pos_v[pl_row, pl.ds(cg * LANES, LANES)]
                plsc.store_scatter(tb_v, [e_rows[cg], ltv, e_subs[cg], lmv],
                                   vals)

        pltpu.async_copy(tb_v.at[:, :, :, pl.ds(0, 128)],
                         out_hbm.at[bb, :, pl.ds(half * NLT, NLT)], osem)

    def pair_body(g2, carry):
        one_chunk(2 * g2, 0)
        one_chunk(2 * g2 + 1, 1)
        return carry

    lax.fori_loop(0, B_PER_W, pair_body, 0)
    pltpu.make_async_copy(tb_v.at[:, :, :, pl.ds(0, 128)],
                          out_hbm.at[0, :, pl.ds(0, NLT)], osem).wait()
    pltpu.make_async_copy(word_hbm.at[pl.ds(0, 2 * CHUNK)], rows_bufs[0],
                          gsems[0]).wait()
    pltpu.make_async_copy(ids_hbm.at[pl.ds(0, CHUNK)], idx_bufs[1],
                          isem).wait()


TBLOCK = 16384


def _tr_body(in_ref, out_ref):
    out_ref[:, :EMB] = in_ref[...].T


_transpose = pl.pallas_call(
    _tr_body,
    grid=(pl.cdiv(VOCAB, TBLOCK),),
    in_specs=[pl.BlockSpec((EMB, TBLOCK), lambda j: (0, j))],
    out_specs=pl.BlockSpec((TBLOCK, 128), lambda j: (j, 0)),
    out_shape=jax.ShapeDtypeStruct((VOCAB, 128), jnp.float32),
)


def kernel(input_ids, word_table, pos_table):
    wt4 = _transpose(word_table.T).reshape(4 * VOCAB, 32)
    ids_flat = input_ids.reshape(N).astype(jnp.int32)
    out5 = _embed(ids_flat, wt4, pos_table)
    return out5.transpose(0, 2, 4, 1, 3).reshape(B, L, EMB)


# speedup vs baseline: 2.1214x; 1.0162x over previous
import functools

import jax
import jax.numpy as jnp
from jax import lax
from jax.experimental import pallas as pl
from jax.experimental.pallas import tpu as pltpu
from jax.experimental.pallas import tpu_sc as plsc

VOCAB = 1000000
EMB = 64
MAXPOS = 512
B = 1024
L = 512

N = B * L
NC = 2
NS = 16
NW = NC * NS
B_PER_W = B // NW
CHUNK = 256
LANES = 16
ET = EMB // 8
NLT = CHUNK // 128
TBW = NLT * 8 * 128

_mesh = plsc.VectorSubcoreMesh(core_axis_name="c", subcore_axis_name="s")


TOT = B_PER_W * 2


def _chunk_start(c, base_b):
    bb = base_b + c // 2
    half = c % 2
    return bb, half, bb * L + half * CHUNK


@functools.partial(
    pl.kernel,
    mesh=_mesh,
    out_type=jax.ShapeDtypeStruct((B, ET, L // 128, 8, 128), jnp.float32),
    scratch_types=[
        pltpu.VMEM((MAXPOS, EMB), jnp.float32),
        pltpu.VMEM((CHUNK,), jnp.int32),
        pltpu.VMEM((CHUNK,), jnp.int32),
        pltpu.VMEM((2 * CHUNK,), jnp.int32),
        pltpu.VMEM((2 * CHUNK,), jnp.int32),
        pltpu.VMEM((2 * CHUNK, 32), jnp.float32),
        pltpu.VMEM((2 * CHUNK, 32), jnp.float32),
        pltpu.VMEM((ET, NLT, 8, 129), jnp.float32),
        pltpu.SemaphoreType.DMA,
        pltpu.SemaphoreType.DMA,
        pltpu.SemaphoreType.DMA,
        pltpu.SemaphoreType.DMA,
    ],
    compiler_params=pltpu.CompilerParams(
        use_tc_tiling_on_sc=False, needs_layout_passes=False,
        disable_bounds_checks=True,
    ),
)
def _embed(ids_hbm, word_hbm, pos_hbm, out_hbm, pos_v, idx0, idx1, pidx0,
           pidx1, rows0, rows1, tb_v, gsem0, gsem1, isem, osem):
    wid = lax.axis_index("s") * NC + lax.axis_index("c")
    base_b = wid * B_PER_W
    idx_bufs = (idx0, idx1)
    pidx_bufs = (pidx0, pidx1)
    rows_bufs = (rows0, rows1)
    gsems = (gsem0, gsem1)

    pltpu.sync_copy(pos_hbm, pos_v)

    viota16 = lax.iota(jnp.int32, LANES)
    e_rows = tuple((jnp.full((LANES,), c * LANES, jnp.int32) + viota16)
                   // 8 for c in range(EMB // LANES))
    e_subs = tuple((jnp.full((LANES,), c * LANES, jnp.int32) + viota16) % 8
                   for c in range(EMB // LANES))


    viota16 = lax.iota(jnp.int32, LANES)
    _srcv = viota16 // 2
    _lowb = viota16 % 2

    def expand_idx(idx_ref, pidx_ref):
        for v in range(2 * CHUNK // LANES):
            vids = plsc.load_gather(idx_ref, [_srcv + v * (LANES // 2)])
            pidx_ref[pl.ds(v * LANES, LANES)] = vids * 4 + _lowb

    _, _, s0 = _chunk_start(0, base_b)
    pltpu.sync_copy(ids_hbm.at[pl.ds(s0, CHUNK)], idx0)
    expand_idx(idx0, pidx0)
    pltpu.async_copy(word_hbm.at[pidx0], rows0, gsem0)
    _, _, s1 = _chunk_start(1, base_b)
    pltpu.async_copy(ids_hbm.at[pl.ds(s1, CHUNK)], idx1, isem)

    def one_chunk(c, p):
        bb, _, _ = _chunk_start(c, base_b)
        half = p
        pltpu.make_async_copy(ids_hbm.at[pl.ds(0, CHUNK)], idx_bufs[1 - p],
                              isem).wait()
        expand_idx(idx_bufs[1 - p], pidx_bufs[1 - p])
        pltpu.async_copy(word_hbm.at[pidx_bufs[1 - p]], rows_bufs[1 - p],
                         gsems[1 - p])
        pltpu.make_async_copy(word_hbm.at[pl.ds(0, 2 * CHUNK)], rows_bufs[p],
                              gsems[p]).wait()
        rows = rows_bufs[p]
        cn2 = jnp.minimum(c + 2, TOT - 1)
        _, _, sn2 = _chunk_start(cn2, base_b)
        pltpu.async_copy(ids_hbm.at[pl.ds(sn2, CHUNK)], idx_bufs[p], isem)

        @pl.when(c > 0)
        def _():
            pltpu.make_async_copy(tb_v.at[:, :, :, pl.ds(0, 128)],
                                  out_hbm.at[0, :, pl.ds(0, NLT)],
                                  osem).wait()

        @plsc.parallel_loop(0, CHUNK, 1, unroll=4)
        def asm_body(l):
            ltv = jnp.full((LANES,), 0, jnp.int32) + l // 128
            lmv = jnp.full((LANES,), 0, jnp.int32) + l % 128
            pl_row = half * CHUNK + l
            for cg in range(EMB // LANES):
                vals = rows[2 * l + cg // 2, pl.ds((cg % 2) * LANES, LANES)]
                vals = vals + pos_v[pl_row, pl.ds(cg * LANES, LANES)]
                plsc.store_scatter(tb_v, [e_rows[cg], ltv, e_subs[cg], lmv],
                                   vals)

        pltpu.async_copy(tb_v.at[:, :, :, pl.ds(0, 128)],
                         out_hbm.at[bb, :, pl.ds(half * NLT, NLT)], osem)

    def pair_body(g2, carry):
        one_chunk(2 * g2, 0)
        one_chunk(2 * g2 + 1, 1)
        return carry

    lax.fori_loop(0, B_PER_W, pair_body, 0)
    pltpu.make_async_copy(tb_v.at[:, :, :, pl.ds(0, 128)],
                          out_hbm.at[0, :, pl.ds(0, NLT)], osem).wait()
    pltpu.make_async_copy(word_hbm.at[pl.ds(0, 2 * CHUNK)], rows_bufs[0],
                          gsems[0]).wait()
    pltpu.make_async_copy(ids_hbm.at[pl.ds(0, CHUNK)], idx_bufs[1],
                          isem).wait()


TBLOCK = 32768


def _tr_body(in_ref, out_ref):
    out_ref[:, :EMB] = in_ref[...].T


_transpose = pl.pallas_call(
    _tr_body,
    grid=(pl.cdiv(VOCAB, TBLOCK),),
    in_specs=[pl.BlockSpec((EMB, TBLOCK), lambda j: (0, j))],
    out_specs=pl.BlockSpec((TBLOCK, 128), lambda j: (j, 0)),
    out_shape=jax.ShapeDtypeStruct((VOCAB, 128), jnp.float32),
)


def kernel(input_ids, word_table, pos_table):
    wt4 = _transpose(word_table.T).reshape(4 * VOCAB, 32)
    ids_flat = input_ids.reshape(N).astype(jnp.int32)
    out5 = _embed(ids_flat, wt4, pos_table)
    return out5.transpose(0, 2, 4, 1, 3).reshape(B, L, EMB)


# speedup vs baseline: 2.1467x; 1.0119x over previous
import functools

import jax
import jax.numpy as jnp
from jax import lax
from jax.experimental import pallas as pl
from jax.experimental.pallas import tpu as pltpu
from jax.experimental.pallas import tpu_sc as plsc

VOCAB = 1000000
EMB = 64
MAXPOS = 512
B = 1024
L = 512

N = B * L
NC = 2
NS = 16
NW = NC * NS
B_PER_W = B // NW
CHUNK = 256
LANES = 16
ET = EMB // 8
NLT = CHUNK // 128
TBW = NLT * 8 * 128

_mesh = plsc.VectorSubcoreMesh(core_axis_name="c", subcore_axis_name="s")


TOT = B_PER_W * 2


def _chunk_start(c, base_b):
    bb = base_b + c // 2
    half = c % 2
    return bb, half, bb * L + half * CHUNK


@functools.partial(
    pl.kernel,
    mesh=_mesh,
    out_type=jax.ShapeDtypeStruct((B, ET, L // 128, 8, 128), jnp.float32),
    scratch_types=[
        pltpu.VMEM((MAXPOS, EMB), jnp.float32),
        pltpu.VMEM((CHUNK,), jnp.int32),
        pltpu.VMEM((CHUNK,), jnp.int32),
        pltpu.VMEM((2 * CHUNK,), jnp.int32),
        pltpu.VMEM((2 * CHUNK,), jnp.int32),
        pltpu.VMEM((2 * CHUNK, 32), jnp.float32),
        pltpu.VMEM((2 * CHUNK, 32), jnp.float32),
        pltpu.VMEM((ET, NLT, 9, 129), jnp.float32),
        pltpu.SemaphoreType.DMA,
        pltpu.SemaphoreType.DMA,
        pltpu.SemaphoreType.DMA,
        pltpu.SemaphoreType.DMA,
    ],
    compiler_params=pltpu.CompilerParams(
        use_tc_tiling_on_sc=False, needs_layout_passes=False,
        disable_bounds_checks=True,
    ),
)
def _embed(ids_hbm, word_hbm, pos_hbm, out_hbm, pos_v, idx0, idx1, pidx0,
           pidx1, rows0, rows1, tb_v, gsem0, gsem1, isem, osem):
    wid = lax.axis_index("s") * NC + lax.axis_index("c")
    base_b = wid * B_PER_W
    idx_bufs = (idx0, idx1)
    pidx_bufs = (pidx0, pidx1)
    rows_bufs = (rows0, rows1)
    gsems = (gsem0, gsem1)

    pltpu.sync_copy(pos_hbm, pos_v)

    viota16 = lax.iota(jnp.int32, LANES)
    e_rows = tuple((jnp.full((LANES,), c * LANES, jnp.int32) + viota16)
                   // 8 for c in range(EMB // LANES))
    e_subs = tuple((jnp.full((LANES,), c * LANES, jnp.int32) + viota16) % 8
                   for c in range(EMB // LANES))


    viota16 = lax.iota(jnp.int32, LANES)
    _srcv = viota16 // 2
    _lowb = viota16 % 2

    def expand_idx(idx_ref, pidx_ref):
        for v in range(2 * CHUNK // LANES):
            vids = plsc.load_gather(idx_ref, [_srcv + v * (LANES // 2)])
            pidx_ref[pl.ds(v * LANES, LANES)] = vids * 4 + _lowb

    _, _, s0 = _chunk_start(0, base_b)
    pltpu.sync_copy(ids_hbm.at[pl.ds(s0, CHUNK)], idx0)
    expand_idx(idx0, pidx0)
    pltpu.async_copy(word_hbm.at[pidx0], rows0, gsem0)
    _, _, s1 = _chunk_start(1, base_b)
    pltpu.async_copy(ids_hbm.at[pl.ds(s1, CHUNK)], idx1, isem)

    def one_chunk(c, p):
        bb, _, _ = _chunk_start(c, base_b)
        half = p
        pltpu.make_async_copy(ids_hbm.at[pl.ds(0, CHUNK)], idx_bufs[1 - p],
                              isem).wait()
        expand_idx(idx_bufs[1 - p], pidx_bufs[1 - p])
        pltpu.async_copy(word_hbm.at[pidx_bufs[1 - p]], rows_bufs[1 - p],
                         gsems[1 - p])
        pltpu.make_async_copy(word_hbm.at[pl.ds(0, 2 * CHUNK)], rows_bufs[p],
                              gsems[p]).wait()
        rows = rows_bufs[p]
        cn2 = jnp.minimum(c + 2, TOT - 1)
        _, _, sn2 = _chunk_start(cn2, base_b)
        pltpu.async_copy(ids_hbm.at[pl.ds(sn2, CHUNK)], idx_bufs[p], isem)

        @pl.when(c > 0)
        def _():
            pltpu.make_async_copy(tb_v.at[:, :, pl.ds(0, 8), pl.ds(0, 128)],
                                  out_hbm.at[0, :, pl.ds(0, NLT)],
                                  osem).wait()

        @plsc.parallel_loop(0, CHUNK, 1, unroll=4)
        def asm_body(l):
            ltv = jnp.full((LANES,), 0, jnp.int32) + l // 128
            lmv = jnp.full((LANES,), 0, jnp.int32) + l % 128
            pl_row = half * CHUNK + l
            for cg in range(EMB // LANES):
                vals = rows[2 * l + cg // 2, pl.ds((cg % 2) * LANES, LANES)]
                vals = vals + pos_v[pl_row, pl.ds(cg * LANES, LANES)]
                plsc.store_scatter(tb_v, [e_rows[cg], ltv, e_subs[cg], lmv],
                                   vals)

        pltpu.async_copy(tb_v.at[:, :, pl.ds(0, 8), pl.ds(0, 128)],
                         out_hbm.at[bb, :, pl.ds(half * NLT, NLT)], osem)

    def pair_body(g2, carry):
        one_chunk(2 * g2, 0)
        one_chunk(2 * g2 + 1, 1)
        return carry

    lax.fori_loop(0, B_PER_W, pair_body, 0)
    pltpu.make_async_copy(tb_v.at[:, :, pl.ds(0, 8), pl.ds(0, 128)],
                          out_hbm.at[0, :, pl.ds(0, NLT)], osem).wait()
    pltpu.make_async_copy(word_hbm.at[pl.ds(0, 2 * CHUNK)], rows_bufs[0],
                          gsems[0]).wait()
    pltpu.make_async_copy(ids_hbm.at[pl.ds(0, CHUNK)], idx_bufs[1],
                          isem).wait()


TBLOCK = 32768


def _tr_body(in_ref, out_ref):
    out_ref[:, :EMB] = in_ref[...].T


_transpose = pl.pallas_call(
    _tr_body,
    grid=(pl.cdiv(VOCAB, TBLOCK),),
    in_specs=[pl.BlockSpec((EMB, TBLOCK), lambda j: (0, j))],
    out_specs=pl.BlockSpec((TBLOCK, 128), lambda j: (j, 0)),
    out_shape=jax.ShapeDtypeStruct((VOCAB, 128), jnp.float32),
)


def kernel(input_ids, word_table, pos_table):
    wt4 = _transpose(word_table.T).reshape(4 * VOCAB, 32)
    ids_flat = input_ids.reshape(N).astype(jnp.int32)
    out5 = _embed(ids_flat, wt4, pos_table)
    return out5.transpose(0, 2, 4, 1, 3).reshape(B, L, EMB)
